# baseline (speedup 1.0000x reference)
import functools

import jax
import jax.numpy as jnp
from jax import lax
from jax.experimental import pallas as pl
from jax.experimental.pallas import tpu as pltpu
from jax.experimental.pallas import tpu_sc as plsc

D = 64
CHUNK = 128
K = 5


@functools.lru_cache(maxsize=None)
def _make(nw, ngrp):
    mesh = plsc.VectorSubcoreMesh(core_axis_name="c", subcore_axis_name="s")
    nc = plsc.get_sparse_core_info().num_cores
    per_w = ngrp * K * CHUNK
    grp_rows = K * CHUNK

    @functools.partial(
        pl.kernel,
        mesh=mesh,
        compiler_params=pltpu.CompilerParams(use_tc_tiling_on_sc=False),
        out_type=jax.ShapeDtypeStruct((nw * per_w, D), jnp.float32),
        scratch_types=[
            pltpu.VMEM((per_w,), jnp.int32),
            pltpu.VMEM((2, grp_rows, D), jnp.float32),
            pltpu.SemaphoreType.DMA,
        ],
    )
    def k(idx_hbm, table_hbm, out_hbm, idx_v, rows_v, gsem):
        wid = lax.axis_index("s") * nc + lax.axis_index("c")
        base = wid * per_w
        pltpu.sync_copy(idx_hbm.at[pl.ds(base, per_w)], idx_v)

        def fire(g, b):
            for kk in range(K):
                pltpu.make_async_copy(
                    table_hbm.at[idx_v.at[pl.ds((g * K + kk) * CHUNK, CHUNK)]],
                    rows_v.at[b, pl.ds(kk * CHUNK, CHUNK)],
                    gsem,
                ).start()

        def drain(b):
            for kk in range(K):
                pltpu.make_async_copy(
                    table_hbm.at[idx_v.at[pl.ds(0, CHUNK)]],
                    rows_v.at[b, pl.ds(kk * CHUNK, CHUNK)],
                    gsem,
                ).wait()

        fire(0, 0)

        def body(i, _):
            for b in range(2):
                g = i * 2 + b

                @pl.when(g + 1 < ngrp)
                def _():
                    fire(g + 1, 1 - b)

                drain(b)
                pltpu.sync_copy(
                    rows_v.at[b], out_hbm.at[pl.ds(base + g * grp_rows, grp_rows)]
                )
            return 0

        lax.fori_loop(0, ngrp // 2, body, 0)

    return k


def kernel(token_ids, weight):
    batch, hist = token_ids.shape
    total = batch * hist
    nw = 32
    grp = nw * CHUNK * K
    assert total % grp == 0 and (total // grp) % 2 == 0
    ngrp = total // grp
    idx = token_ids.reshape(total).astype(jnp.int32)
    out = _make(nw, ngrp)(idx, weight)
    return out.reshape(batch, hist, D)
